# Initial kernel scaffold; baseline (speedup 1.0000x reference)
import jax
import jax.numpy as jnp
from jax.experimental import pallas as pl


def kernel(pos_indicator, predicts, gt_loc, gt_conf):
    raise NotImplementedError("write your pallas kernel here")


# speedup vs baseline: 4.3527x; 4.3527x over previous
import jax
import jax.numpy as jnp
from jax.experimental import pallas as pl
from jax.experimental.pallas import tpu as pltpu

B, D, C = 32, 8732, 21
ALPHA = 1.0
NEG_FACTOR = 3.0
N_ITERS = 28


def _ssd_step(pos_ref, xloc_ref, xconf_ref, gloc_ref, gconf_ref, out_ref,
              pn_s, l1p_s, pos_s):
    b = pl.program_id(0)

    pos = pos_ref[0]
    xloc = xloc_ref[0]
    xconf = xconf_ref[0]
    gloc = gloc_ref[0]
    gconf = gconf_ref[0]

    d = xloc - gloc
    ad = jnp.abs(d)
    sl1 = jnp.where(ad < 1.0, 0.5 * d * d, ad - 0.5)
    l1 = jnp.sum(sl1, axis=0, keepdims=True)

    e = jnp.exp(xconf)
    lse = jnp.log(jnp.sum(e, axis=0, keepdims=True))
    gsum = jnp.sum(gconf, axis=0, keepdims=True)
    dot = jnp.sum(gconf * xconf, axis=0, keepdims=True)
    pn = gsum * lse - dot

    pn_s[pl.ds(b, 1), :] = pn
    l1p_s[pl.ds(b, 1), :] = l1 * pos
    pos_s[pl.ds(b, 1), :] = pos

    @pl.when(b == B - 1)
    def _final():
        pn_all = pn_s[:, :]
        posa = pos_s[:, :]
        l1pa = l1p_s[:, :]

        n_pos = jnp.sum(posa, axis=1, keepdims=True)
        p_sum = jnp.sum(pn_all * posa, axis=1, keepdims=True)
        l1_sum = jnp.sum(l1pa, axis=1, keepdims=True)
        neg_cnt = jnp.float32(D) - n_pos
        k = jnp.minimum(neg_cnt, NEG_FACTOR * n_pos)

        negv = jnp.where(posa > 0.5, -1.0, pn_all)

        lo0 = jnp.full((B, 1), -0.5, jnp.float32)
        hi0 = jnp.max(negv, axis=1, keepdims=True) + 1.0

        def body(_, carry):
            lo, hi = carry
            mid = 0.5 * (lo + hi)
            cnt = jnp.sum(jnp.where(negv > mid, 1.0, 0.0), axis=1,
                          keepdims=True)
            ge = cnt >= k
            return jnp.where(ge, mid, lo), jnp.where(ge, hi, mid)

        lo, hi = jax.lax.fori_loop(0, N_ITERS, body, (lo0, hi0))
        gt = negv > hi
        c = jnp.sum(jnp.where(gt, 1.0, 0.0), axis=1, keepdims=True)
        sum_gt = jnp.sum(jnp.where(gt, negv, 0.0), axis=1, keepdims=True)
        n_sum = sum_gt + (k - c) * hi
        n_sum = jnp.where(k > 0.0, n_sum, 0.0)

        safe_n = jnp.maximum(n_pos, 1.0)
        has_pos = n_pos > 0.0
        conf_loss = jnp.where(has_pos, (p_sum + n_sum) / safe_n, 0.0)
        loc_loss = jnp.where(has_pos, l1_sum / safe_n, 0.0)
        total = jnp.sum(conf_loss + ALPHA * loc_loss, axis=0,
                        keepdims=True) / jnp.float32(B)
        out_ref[:, :] = total


def kernel(pos_indicator, predicts, gt_loc, gt_conf):
    posf = pos_indicator.astype(jnp.float32)[:, None, :]
    xloc = jnp.transpose(predicts[:, :, :4], (0, 2, 1))
    xconf = jnp.transpose(predicts[:, :, 4:], (0, 2, 1))
    gloc_t = jnp.transpose(gt_loc, (0, 2, 1))
    gconf_t = jnp.transpose(gt_conf, (0, 2, 1))

    out = pl.pallas_call(
        _ssd_step,
        grid=(B,),
        in_specs=[
            pl.BlockSpec((1, 1, D), lambda b: (b, 0, 0)),
            pl.BlockSpec((1, 4, D), lambda b: (b, 0, 0)),
            pl.BlockSpec((1, C, D), lambda b: (b, 0, 0)),
            pl.BlockSpec((1, 4, D), lambda b: (b, 0, 0)),
            pl.BlockSpec((1, C, D), lambda b: (b, 0, 0)),
        ],
        out_specs=pl.BlockSpec((1, 1), lambda b: (0, 0)),
        out_shape=jax.ShapeDtypeStruct((1, 1), jnp.float32),
        scratch_shapes=[
            pltpu.VMEM((B, D), jnp.float32),
            pltpu.VMEM((B, D), jnp.float32),
            pltpu.VMEM((B, D), jnp.float32),
        ],
    )(posf, xloc, xconf, gloc_t, gconf_t)
    return out[0, 0]
